# baseline (speedup 1.0000x reference)
import functools

import jax
import jax.numpy as jnp
from jax.experimental import pallas as pl

_ALPHA = 0.25
_BN = 10000


def _fneg(x):
    return (1.0 - _ALPHA) * x * x * (-jnp.log(1.0 - x))


def _fpos(x):
    return _ALPHA * (1.0 - x) * (1.0 - x) * (-jnp.log(x))


def _body(cls_ref, regt_ref, anct_ref, ann_ref, out_ref, *, n_ann):
    i = pl.program_id(1)

    ann = ann_ref[0]
    anct = anct_ref[0].T
    regt = regt_ref[0].T
    cls = jnp.clip(cls_ref[0], 1e-4, 1.0 - 1e-4)

    ax0, ay0, ax1, ay1 = anct[0], anct[1], anct[2], anct[3]
    aw = ax1 - ax0
    ah = ay1 - ay0
    acx = ax0 + 0.5 * aw
    acy = ay0 + 0.5 * ah

    b0 = ann[:, 0:1]
    b1 = ann[:, 1:2]
    b2 = ann[:, 2:3]
    b3 = ann[:, 3:4]
    area_b = (b2 - b0) * (b3 - b1)
    iw = jnp.minimum(ax1[None, :], b2) - jnp.maximum(ax0[None, :], b0)
    ih = jnp.minimum(ay1[None, :], b3) - jnp.maximum(ay0[None, :], b1)
    iw = jnp.maximum(iw, 0.0)
    ih = jnp.maximum(ih, 0.0)
    inter = iw * ih
    ua = jnp.maximum((aw * ah)[None, :] + area_b - inter, 1e-8)
    iou = inter / ua

    iou_max = jnp.max(iou, axis=0)
    m_iota = jax.lax.broadcasted_iota(jnp.int32, iou.shape, 0)
    argmax = jnp.min(jnp.where(iou == iou_max[None, :], m_iota, n_ann), axis=0)
    onehot_m = (m_iota == argmax[None, :]).astype(jnp.float32)
    assigned = jax.lax.dot_general(ann, onehot_m, (((0,), (0,)), ((), ())),
                                   preferred_element_type=jnp.float32)

    pos = iou_max >= 0.5
    neg = iou_max < 0.4
    posf = pos.astype(jnp.float32)
    num_pos = jnp.sum(posf)

    fmat = _fneg(cls)
    w_row = (posf + neg.astype(jnp.float32)).reshape(1, -1)
    base = jnp.sum(jnp.dot(w_row, fmat, preferred_element_type=jnp.float32))

    k = fmat.shape[1]
    ci = ann[:, 4:5].astype(jnp.int32)
    conehot = (ci == jax.lax.broadcasted_iota(jnp.int32, (ci.shape[0], k), 1)
               ).astype(jnp.float32)
    dt = jax.lax.dot_general(conehot, cls, (((1,), (1,)), ((), ())),
                             preferred_element_type=jnp.float32)
    x_c = jnp.sum(dt * onehot_m, axis=0)
    li = assigned[4].astype(jnp.int32)
    has_c = (li >= 0) & (li < k)
    corr = jnp.where(has_c, _fpos(x_c) - _fneg(x_c), 0.0)
    cls_sum = base + jnp.sum(posf * corr)

    gw = assigned[2] - assigned[0]
    gh = assigned[3] - assigned[1]
    gcx = assigned[0] + 0.5 * gw
    gcy = assigned[1] + 0.5 * gh
    gw = jnp.maximum(gw, 1.0)
    gh = jnp.maximum(gh, 1.0)
    tdx = (gcx - acx) / aw / 0.1
    tdy = (gcy - acy) / ah / 0.1
    tdw = jnp.log(gw / aw) / 0.2
    tdh = jnp.log(gh / ah) / 0.2

    def _sl1(t, r):
        d = jnp.abs(t - r)
        return jnp.where(d <= 1.0 / 9.0, 4.5 * d * d, d - 0.5 / 9.0)

    rl = (_sl1(tdx, regt[0]) + _sl1(tdy, regt[1])
          + _sl1(tdw, regt[2]) + _sl1(tdh, regt[3]))
    reg_sum = jnp.sum(rl * posf)

    lane = jax.lax.broadcasted_iota(jnp.int32, (1, 1, 8), 2)
    part = (jnp.where(lane == 0, cls_sum, 0.0)
            + jnp.where(lane == 1, num_pos, 0.0)
            + jnp.where(lane == 2, reg_sum, 0.0))

    @pl.when(i == 0)
    def _init():
        out_ref[...] = part

    @pl.when(i != 0)
    def _acc():
        out_ref[...] += part


def kernel(classifications, regressions, anchors, annotations):
    B, N, K = classifications.shape
    M = annotations.shape[1]
    nb = N // _BN

    partials = pl.pallas_call(
        functools.partial(_body, n_ann=M),
        grid=(B, nb),
        in_specs=[
            pl.BlockSpec((1, _BN, K), lambda j, i: (j, i, 0)),
            pl.BlockSpec((1, _BN, 4), lambda j, i: (j, i, 0)),
            pl.BlockSpec((1, _BN, 4), lambda j, i: (0, i, 0)),
            pl.BlockSpec((1, M, 5), lambda j, i: (j, 0, 0)),
        ],
        out_specs=pl.BlockSpec((1, 1, 8), lambda j, i: (j, 0, 0)),
        out_shape=jax.ShapeDtypeStruct((B, 1, 8), jnp.float32),
    )(classifications, regressions, anchors, annotations)

    cls_sum = partials[:, 0, 0]
    num_pos = partials[:, 0, 1]
    reg_sum = partials[:, 0, 2]
    cls_loss = cls_sum / jnp.maximum(num_pos, 1.0)
    reg_loss = jnp.where(num_pos > 0, reg_sum / jnp.maximum(num_pos * 4.0, 1.0), 0.0)
    return (jnp.mean(cls_loss, keepdims=True), jnp.mean(reg_loss, keepdims=True))


# speedup vs baseline: 2.0833x; 2.0833x over previous
import functools

import jax
import jax.numpy as jnp
from jax.experimental import pallas as pl

_ALPHA = 0.25
_BN = 4096


def _fneg(x):
    return (1.0 - _ALPHA) * x * x * (-jnp.log(1.0 - x))


def _fpos(x):
    return _ALPHA * (1.0 - x) * (1.0 - x) * (-jnp.log(x))


def _body(clst_ref, regt_ref, anct_ref, ann_ref, out_ref, *, n_ann, n_total):
    i = pl.program_id(1)

    ann = ann_ref[0]
    anct = anct_ref[0]
    regt = regt_ref[0]
    clst = clst_ref[0]
    k = clst.shape[0]

    valid2 = (jax.lax.broadcasted_iota(jnp.int32, (1, clst.shape[1]), 1)
              + i * clst.shape[1]) < n_total
    valid = valid2[0]
    validf = valid.astype(jnp.float32)

    cls = jnp.clip(jnp.where(valid2, clst, 0.5), 1e-4, 1.0 - 1e-4)

    ax0, ay0, ax1, ay1 = anct[0], anct[1], anct[2], anct[3]
    aw = ax1 - ax0
    ah = ay1 - ay0
    acx = ax0 + 0.5 * aw
    acy = ay0 + 0.5 * ah

    b0 = ann[:, 0:1]
    b1 = ann[:, 1:2]
    b2 = ann[:, 2:3]
    b3 = ann[:, 3:4]
    area_b = (b2 - b0) * (b3 - b1)
    iw = jnp.minimum(ax1[None, :], b2) - jnp.maximum(ax0[None, :], b0)
    ih = jnp.minimum(ay1[None, :], b3) - jnp.maximum(ay0[None, :], b1)
    iw = jnp.maximum(iw, 0.0)
    ih = jnp.maximum(ih, 0.0)
    inter = iw * ih
    ua = jnp.maximum((aw * ah)[None, :] + area_b - inter, 1e-8)
    iou = inter / ua

    iou_max = jnp.max(iou, axis=0)
    m_iota = jax.lax.broadcasted_iota(jnp.int32, iou.shape, 0)
    argmax = jnp.min(jnp.where(iou == iou_max[None, :], m_iota, n_ann), axis=0)
    onehot_m = (m_iota == argmax[None, :]).astype(jnp.float32)
    assigned = jax.lax.dot_general(ann, onehot_m, (((0,), (0,)), ((), ())),
                                   preferred_element_type=jnp.float32)

    pos = iou_max >= 0.5
    neg = iou_max < 0.4
    posf = pos.astype(jnp.float32)
    num_pos = jnp.sum(posf * validf)

    row_fneg = jnp.sum(_fneg(cls), axis=0)
    w = (posf + neg.astype(jnp.float32)) * validf
    base = jnp.sum(row_fneg * w)

    ci = ann[:, 4:5].astype(jnp.int32)
    conehot = (ci == jax.lax.broadcasted_iota(jnp.int32, (ci.shape[0], k), 1)
               ).astype(jnp.float32)
    dt = jnp.dot(conehot, cls, preferred_element_type=jnp.float32)
    x_c = jnp.sum(dt * onehot_m, axis=0)
    li = assigned[4].astype(jnp.int32)
    has_c = (li >= 0) & (li < k) & valid
    corr = jnp.where(has_c, _fpos(x_c) - _fneg(x_c), 0.0)
    cls_sum = base + jnp.sum(posf * corr)

    gw = assigned[2] - assigned[0]
    gh = assigned[3] - assigned[1]
    gcx = assigned[0] + 0.5 * gw
    gcy = assigned[1] + 0.5 * gh
    gw = jnp.maximum(gw, 1.0)
    gh = jnp.maximum(gh, 1.0)
    tdx = (gcx - acx) / aw / 0.1
    tdy = (gcy - acy) / ah / 0.1
    tdw = jnp.log(gw / aw) / 0.2
    tdh = jnp.log(gh / ah) / 0.2

    def _sl1(t, r):
        d = jnp.abs(t - r)
        return jnp.where(d <= 1.0 / 9.0, 4.5 * d * d, d - 0.5 / 9.0)

    rl = (_sl1(tdx, regt[0]) + _sl1(tdy, regt[1])
          + _sl1(tdw, regt[2]) + _sl1(tdh, regt[3]))
    reg_sum = jnp.sum(jnp.where(valid, rl * posf, 0.0))

    lane = jax.lax.broadcasted_iota(jnp.int32, (1, 1, 8), 2)
    part = (jnp.where(lane == 0, cls_sum, 0.0)
            + jnp.where(lane == 1, num_pos, 0.0)
            + jnp.where(lane == 2, reg_sum, 0.0))

    @pl.when(i == 0)
    def _init():
        out_ref[...] = part

    @pl.when(i != 0)
    def _acc():
        out_ref[...] += part


def kernel(classifications, regressions, anchors, annotations):
    B, N, K = classifications.shape
    M = annotations.shape[1]
    nb = pl.cdiv(N, _BN)

    clst = jnp.transpose(classifications, (0, 2, 1))
    regt = jnp.transpose(regressions, (0, 2, 1))
    anct = jnp.transpose(anchors, (0, 2, 1))

    partials = pl.pallas_call(
        functools.partial(_body, n_ann=M, n_total=N),
        grid=(B, nb),
        in_specs=[
            pl.BlockSpec((1, K, _BN), lambda j, i: (j, 0, i)),
            pl.BlockSpec((1, 4, _BN), lambda j, i: (j, 0, i)),
            pl.BlockSpec((1, 4, _BN), lambda j, i: (0, 0, i)),
            pl.BlockSpec((1, M, 5), lambda j, i: (j, 0, 0)),
        ],
        out_specs=pl.BlockSpec((1, 1, 8), lambda j, i: (j, 0, 0)),
        out_shape=jax.ShapeDtypeStruct((B, 1, 8), jnp.float32),
    )(clst, regt, anct, annotations)

    cls_sum = partials[:, 0, 0]
    num_pos = partials[:, 0, 1]
    reg_sum = partials[:, 0, 2]
    cls_loss = cls_sum / jnp.maximum(num_pos, 1.0)
    reg_loss = jnp.where(num_pos > 0, reg_sum / jnp.maximum(num_pos * 4.0, 1.0), 0.0)
    return (jnp.mean(cls_loss, keepdims=True), jnp.mean(reg_loss, keepdims=True))


# speedup vs baseline: 2.1675x; 1.0404x over previous
import functools

import jax
import jax.numpy as jnp
from jax.experimental import pallas as pl

_ALPHA = 0.25
_BN = 8192


def _fneg(x):
    return (1.0 - _ALPHA) * x * x * (-jnp.log(1.0 - x))


def _fpos(x):
    return _ALPHA * (1.0 - x) * (1.0 - x) * (-jnp.log(x))


def _body(clst_ref, regt_ref, anct_ref, ann_ref, out_ref, *, n_ann, n_total):
    i = pl.program_id(1)

    ann = ann_ref[0]
    anct = anct_ref[0]
    regt = regt_ref[0]
    clst = clst_ref[0]
    k = clst.shape[0]

    valid2 = (jax.lax.broadcasted_iota(jnp.int32, (1, clst.shape[1]), 1)
              + i * clst.shape[1]) < n_total
    valid = valid2[0]
    validf = valid.astype(jnp.float32)

    cls = jnp.clip(jnp.where(valid2, clst, 0.5), 1e-4, 1.0 - 1e-4)

    ax0, ay0, ax1, ay1 = anct[0], anct[1], anct[2], anct[3]
    aw = ax1 - ax0
    ah = ay1 - ay0
    acx = ax0 + 0.5 * aw
    acy = ay0 + 0.5 * ah

    b0 = ann[:, 0:1]
    b1 = ann[:, 1:2]
    b2 = ann[:, 2:3]
    b3 = ann[:, 3:4]
    area_b = (b2 - b0) * (b3 - b1)
    iw = jnp.minimum(ax1[None, :], b2) - jnp.maximum(ax0[None, :], b0)
    ih = jnp.minimum(ay1[None, :], b3) - jnp.maximum(ay0[None, :], b1)
    iw = jnp.maximum(iw, 0.0)
    ih = jnp.maximum(ih, 0.0)
    inter = iw * ih
    ua = jnp.maximum((aw * ah)[None, :] + area_b - inter, 1e-8)
    iou = inter / ua

    iou_max = jnp.max(iou, axis=0)
    m_iota = jax.lax.broadcasted_iota(jnp.int32, iou.shape, 0)
    argmax = jnp.min(jnp.where(iou == iou_max[None, :], m_iota, n_ann), axis=0)
    onehot_m = (m_iota == argmax[None, :]).astype(jnp.float32)
    assigned = jax.lax.dot_general(ann, onehot_m, (((0,), (0,)), ((), ())),
                                   preferred_element_type=jnp.float32)

    pos = iou_max >= 0.5
    neg = iou_max < 0.4
    posf = pos.astype(jnp.float32)
    num_pos = jnp.sum(posf * validf)

    row_fneg = jnp.sum(_fneg(cls), axis=0)
    w = (posf + neg.astype(jnp.float32)) * validf
    base = jnp.sum(row_fneg * w)

    ci = ann[:, 4:5].astype(jnp.int32)
    conehot = (ci == jax.lax.broadcasted_iota(jnp.int32, (ci.shape[0], k), 1)
               ).astype(jnp.float32)
    dt = jnp.dot(conehot, cls, preferred_element_type=jnp.float32)
    x_c = jnp.sum(dt * onehot_m, axis=0)
    li = assigned[4].astype(jnp.int32)
    has_c = (li >= 0) & (li < k) & valid
    corr = jnp.where(has_c, _fpos(x_c) - _fneg(x_c), 0.0)
    cls_sum = base + jnp.sum(posf * corr)

    gw = assigned[2] - assigned[0]
    gh = assigned[3] - assigned[1]
    gcx = assigned[0] + 0.5 * gw
    gcy = assigned[1] + 0.5 * gh
    gw = jnp.maximum(gw, 1.0)
    gh = jnp.maximum(gh, 1.0)
    tdx = (gcx - acx) / aw / 0.1
    tdy = (gcy - acy) / ah / 0.1
    tdw = jnp.log(gw / aw) / 0.2
    tdh = jnp.log(gh / ah) / 0.2

    def _sl1(t, r):
        d = jnp.abs(t - r)
        return jnp.where(d <= 1.0 / 9.0, 4.5 * d * d, d - 0.5 / 9.0)

    rl = (_sl1(tdx, regt[0]) + _sl1(tdy, regt[1])
          + _sl1(tdw, regt[2]) + _sl1(tdh, regt[3]))
    reg_sum = jnp.sum(jnp.where(valid, rl * posf, 0.0))

    lane = jax.lax.broadcasted_iota(jnp.int32, (1, 1, 8), 2)
    part = (jnp.where(lane == 0, cls_sum, 0.0)
            + jnp.where(lane == 1, num_pos, 0.0)
            + jnp.where(lane == 2, reg_sum, 0.0))

    @pl.when(i == 0)
    def _init():
        out_ref[...] = part

    @pl.when(i != 0)
    def _acc():
        out_ref[...] += part


def kernel(classifications, regressions, anchors, annotations):
    B, N, K = classifications.shape
    M = annotations.shape[1]
    nb = pl.cdiv(N, _BN)

    clst = jnp.transpose(classifications, (0, 2, 1))
    regt = jnp.transpose(regressions, (0, 2, 1))
    anct = jnp.transpose(anchors, (0, 2, 1))

    partials = pl.pallas_call(
        functools.partial(_body, n_ann=M, n_total=N),
        grid=(B, nb),
        in_specs=[
            pl.BlockSpec((1, K, _BN), lambda j, i: (j, 0, i)),
            pl.BlockSpec((1, 4, _BN), lambda j, i: (j, 0, i)),
            pl.BlockSpec((1, 4, _BN), lambda j, i: (0, 0, i)),
            pl.BlockSpec((1, M, 5), lambda j, i: (j, 0, 0)),
        ],
        out_specs=pl.BlockSpec((1, 1, 8), lambda j, i: (j, 0, 0)),
        out_shape=jax.ShapeDtypeStruct((B, 1, 8), jnp.float32),
    )(clst, regt, anct, annotations)

    cls_sum = partials[:, 0, 0]
    num_pos = partials[:, 0, 1]
    reg_sum = partials[:, 0, 2]
    cls_loss = cls_sum / jnp.maximum(num_pos, 1.0)
    reg_loss = jnp.where(num_pos > 0, reg_sum / jnp.maximum(num_pos * 4.0, 1.0), 0.0)
    return (jnp.mean(cls_loss, keepdims=True), jnp.mean(reg_loss, keepdims=True))


# speedup vs baseline: 2.2512x; 1.0386x over previous
import functools

import jax
import jax.numpy as jnp
from jax.experimental import pallas as pl

_ALPHA = 0.25
_BN = 8192


def _fneg(x):
    return (1.0 - _ALPHA) * x * x * (-jnp.log(1.0 - x))


def _fpos(x):
    return _ALPHA * (1.0 - x) * (1.0 - x) * (-jnp.log(x))


def _body(clst_ref, regt_ref, anct_ref, ann_ref, out_ref, *, n_ann, n_total):
    i = pl.program_id(1)

    ann = ann_ref[0]
    anct = anct_ref[0]
    regt = regt_ref[0]
    clst = clst_ref[0]
    k = clst.shape[0]

    valid2 = (jax.lax.broadcasted_iota(jnp.int32, (1, clst.shape[1]), 1)
              + i * clst.shape[1]) < n_total
    valid = valid2[0]
    validf = valid.astype(jnp.float32)

    cls = jnp.clip(jnp.where(valid2, clst, 0.5), 1e-4, 1.0 - 1e-4)

    ax0, ay0, ax1, ay1 = anct[0], anct[1], anct[2], anct[3]
    aw = ax1 - ax0
    ah = ay1 - ay0
    acx = ax0 + 0.5 * aw
    acy = ay0 + 0.5 * ah

    b0 = ann[:, 0:1]
    b1 = ann[:, 1:2]
    b2 = ann[:, 2:3]
    b3 = ann[:, 3:4]
    area_b = (b2 - b0) * (b3 - b1)
    iw = jnp.minimum(ax1[None, :], b2) - jnp.maximum(ax0[None, :], b0)
    ih = jnp.minimum(ay1[None, :], b3) - jnp.maximum(ay0[None, :], b1)
    iw = jnp.maximum(iw, 0.0)
    ih = jnp.maximum(ih, 0.0)
    inter = iw * ih
    ua = jnp.maximum((aw * ah)[None, :] + area_b - inter, 1e-8)
    iou = inter / ua

    iou_max = jnp.max(iou, axis=0)
    m_iota = jax.lax.broadcasted_iota(jnp.int32, iou.shape, 0)
    argmax = jnp.min(jnp.where(iou == iou_max[None, :], m_iota, n_ann), axis=0)
    onehot_m = (m_iota == argmax[None, :]).astype(jnp.float32)
    assigned = jax.lax.dot_general(ann, onehot_m, (((0,), (0,)), ((), ())),
                                   preferred_element_type=jnp.float32)

    pos = iou_max >= 0.5
    neg = iou_max < 0.4
    posf = pos.astype(jnp.float32)
    num_pos = jnp.sum(posf * validf)

    fmat = _fneg(cls)
    ones_k = jnp.full((1, k), 1.0, dtype=jnp.float32)
    row_fneg = jnp.dot(ones_k, fmat,
                       preferred_element_type=jnp.float32)[0]
    w = (posf + neg.astype(jnp.float32)) * validf
    base = jnp.sum(row_fneg * w)

    ci = ann[:, 4:5].astype(jnp.int32)
    conehot = (ci == jax.lax.broadcasted_iota(jnp.int32, (ci.shape[0], k), 1)
               ).astype(jnp.float32)
    dt = jnp.dot(conehot, cls, preferred_element_type=jnp.float32)
    ones_m = jnp.full((1, dt.shape[0]), 1.0, dtype=jnp.float32)
    x_c = jnp.dot(ones_m, dt * onehot_m,
                  preferred_element_type=jnp.float32)[0]
    li = assigned[4].astype(jnp.int32)
    has_c = (li >= 0) & (li < k) & valid
    corr = jnp.where(has_c, _fpos(x_c) - _fneg(x_c), 0.0)
    cls_sum = base + jnp.sum(posf * corr)

    gw = assigned[2] - assigned[0]
    gh = assigned[3] - assigned[1]
    gcx = assigned[0] + 0.5 * gw
    gcy = assigned[1] + 0.5 * gh
    gw = jnp.maximum(gw, 1.0)
    gh = jnp.maximum(gh, 1.0)
    tdx = (gcx - acx) / aw / 0.1
    tdy = (gcy - acy) / ah / 0.1
    tdw = jnp.log(gw / aw) / 0.2
    tdh = jnp.log(gh / ah) / 0.2

    def _sl1(t, r):
        d = jnp.abs(t - r)
        return jnp.where(d <= 1.0 / 9.0, 4.5 * d * d, d - 0.5 / 9.0)

    rl = (_sl1(tdx, regt[0]) + _sl1(tdy, regt[1])
          + _sl1(tdw, regt[2]) + _sl1(tdh, regt[3]))
    reg_sum = jnp.sum(jnp.where(valid, rl * posf, 0.0))

    lane = jax.lax.broadcasted_iota(jnp.int32, (1, 1, 8), 2)
    part = (jnp.where(lane == 0, cls_sum, 0.0)
            + jnp.where(lane == 1, num_pos, 0.0)
            + jnp.where(lane == 2, reg_sum, 0.0))

    @pl.when(i == 0)
    def _init():
        out_ref[...] = part

    @pl.when(i != 0)
    def _acc():
        out_ref[...] += part


def kernel(classifications, regressions, anchors, annotations):
    B, N, K = classifications.shape
    M = annotations.shape[1]
    nb = pl.cdiv(N, _BN)

    clst = jnp.transpose(classifications, (0, 2, 1))
    regt = jnp.transpose(regressions, (0, 2, 1))
    anct = jnp.transpose(anchors, (0, 2, 1))

    partials = pl.pallas_call(
        functools.partial(_body, n_ann=M, n_total=N),
        grid=(B, nb),
        in_specs=[
            pl.BlockSpec((1, K, _BN), lambda j, i: (j, 0, i)),
            pl.BlockSpec((1, 4, _BN), lambda j, i: (j, 0, i)),
            pl.BlockSpec((1, 4, _BN), lambda j, i: (0, 0, i)),
            pl.BlockSpec((1, M, 5), lambda j, i: (j, 0, 0)),
        ],
        out_specs=pl.BlockSpec((1, 1, 8), lambda j, i: (j, 0, 0)),
        out_shape=jax.ShapeDtypeStruct((B, 1, 8), jnp.float32),
    )(clst, regt, anct, annotations)

    cls_sum = partials[:, 0, 0]
    num_pos = partials[:, 0, 1]
    reg_sum = partials[:, 0, 2]
    cls_loss = cls_sum / jnp.maximum(num_pos, 1.0)
    reg_loss = jnp.where(num_pos > 0, reg_sum / jnp.maximum(num_pos * 4.0, 1.0), 0.0)
    return (jnp.mean(cls_loss, keepdims=True), jnp.mean(reg_loss, keepdims=True))
